# baseline (speedup 1.0000x reference)
import jax
import jax.numpy as jnp
from jax import lax
from jax.experimental import pallas as pl
from jax.experimental.pallas import tpu as pltpu
from jax.experimental.pallas import tpu_sc as plsc

N = 10000
E = 320000
D = 128
ED = 16

NC = 2
NS = 16
NW = NC * NS
L = 16

EPW = E // NW
BLK = 2000
NBLK = EPW // BLK
GRP = BLK // L
RING = 5

NPAD = 10240
STRIPE = NPAD // NS

EB = 3200


def _node_alpha_body(h_ref, wa_ref, o1_ref, o2_ref):
    hb = h_ref[...]
    w1 = wa_ref[0, :D]
    w2 = wa_ref[0, D:2 * D]
    o1_ref[...] = jnp.dot(hb, w1, preferred_element_type=jnp.float32)
    o2_ref[...] = jnp.dot(hb, w2, preferred_element_type=jnp.float32)


def _node_alpha(h, W_att):
    return pl.pallas_call(
        _node_alpha_body,
        out_shape=[
            jax.ShapeDtypeStruct((N,), jnp.float32),
            jax.ShapeDtypeStruct((N,), jnp.float32),
        ],
    )(h, W_att)


def _edge_lite_body(ei_ref, e_ref, we_ref, wa_ref, src_ref, dst_ref, b_ref):
    i = pl.program_id(0)
    sl = pl.ds(i * EB, EB)
    src_ref[sl] = ei_ref[0, :]
    dst_ref[sl] = ei_ref[1, :]
    w3 = wa_ref[0, 2 * D:]
    v3 = lax.dot_general(w3, we_ref[...], (((0,), (0,)), ((), ())),
                         preferred_element_type=jnp.float32)
    b_ref[sl] = jnp.dot(e_ref[...], v3, preferred_element_type=jnp.float32)


def _edge_lite(edge_index, e, W_edge, W_att):
    return pl.pallas_call(
        _edge_lite_body,
        grid=(E // EB,),
        in_specs=[
            pl.BlockSpec((2, EB), lambda i: (0, i)),
            pl.BlockSpec((EB, ED), lambda i: (i, 0)),
            pl.BlockSpec((ED, ED), lambda i: (0, 0)),
            pl.BlockSpec((1, 2 * D + ED), lambda i: (0, 0)),
        ],
        out_specs=[
            pl.BlockSpec((E,), lambda i: (0,)),
            pl.BlockSpec((E,), lambda i: (0,)),
            pl.BlockSpec((E,), lambda i: (0,)),
        ],
        out_shape=[
            jax.ShapeDtypeStruct((E,), jnp.int32),
            jax.ShapeDtypeStruct((E,), jnp.int32),
            jax.ShapeDtypeStruct((E,), jnp.float32),
        ],
    )(edge_index, e, W_edge, W_att)


def _sc_body(src_hbm, dst_hbm, asrc_hbm, adst_hbm, beta_hbm, e_hbm,
             s_out, den_out,
             asrc_v, adst_v, den_v, src_v, dstf_v, beta_v, e_v,
             rows0, rows1, rows2, rows3, rows4, zbuf_v, dsum_v,
             s_sh, den_sh, insem, sem0, sem1, sem2, sem3, sem4):
    cid = lax.axis_index("c")
    sid = lax.axis_index("s")
    wid = cid * NS + sid
    ebase = wid * EPW

    ring_bufs = (rows0, rows1, rows2, rows3, rows4)
    ring_sems = (sem0, sem1, sem2, sem3, sem4)
    zero16 = jnp.zeros((L,), jnp.float32)
    dummy16 = e_hbm.at[pl.ds(0, L), :]

    pltpu.async_copy(asrc_hbm, asrc_v, insem)
    pltpu.async_copy(adst_hbm, adst_v, insem)
    pltpu.make_async_copy(asrc_hbm, asrc_v, insem).wait()
    pltpu.make_async_copy(adst_hbm, adst_v, insem).wait()

    def _z_buf(i, _):
        zbuf_v[i, :] = zero16
        return _
    lax.fori_loop(0, STRIPE, _z_buf, None)

    def _z_den(i, _):
        den_v[pl.ds(i * L, L)] = zero16
        return _
    lax.fori_loop(0, NPAD // L, _z_den, None)

    pltpu.sync_copy(zbuf_v, s_sh.at[pl.ds(sid * STRIPE, STRIPE)])
    plsc.subcore_barrier()

    for blk in range(NBLK):
        base = ebase + blk * BLK
        pltpu.async_copy(src_hbm.at[pl.ds(base, BLK)], src_v, insem)
        pltpu.async_copy(dst_hbm.at[pl.ds(base, BLK)], dstf_v, insem)
        pltpu.async_copy(beta_hbm.at[pl.ds(base, BLK)], beta_v, insem)
        pltpu.async_copy(e_hbm.at[pl.ds(base, BLK), :], e_v, insem)
        pltpu.make_async_copy(src_hbm.at[pl.ds(base, BLK)], src_v, insem).wait()
        pltpu.make_async_copy(dst_hbm.at[pl.ds(base, BLK)], dstf_v, insem).wait()
        pltpu.make_async_copy(beta_hbm.at[pl.ds(base, BLK)], beta_v, insem).wait()
        pltpu.make_async_copy(e_hbm.at[pl.ds(base, BLK), :], e_v, insem).wait()

        def _outer(g2, _):
            for k in range(RING):
                g = g2 * RING + k
                o = g * L
                sv = src_v[pl.ds(o, L)]
                dv = dstf_v[pl.ds(o, L)]
                a1 = plsc.load_gather(asrc_v, [sv])
                a2 = plsc.load_gather(adst_v, [dv])
                att = a1 + a2 + beta_v[pl.ds(o, L)]
                att = jnp.maximum(att, att * 0.01)
                ee = jnp.exp(att)
                plsc.addupdate_scatter(den_v, [dv], ee)
                buf, sem = ring_bufs[k], ring_sems[k]
                if blk == 0:
                    @pl.when(g2 >= 1)
                    def _drain():
                        pltpu.make_async_copy(dummy16, buf, sem).wait()
                else:
                    pltpu.make_async_copy(dummy16, buf, sem).wait()
                for j in range(L):
                    buf[j, :] = e_v[o + j, :] * ee[j]
                pltpu.async_copy(buf, s_sh.at[dv], sem, add=True)
            return _
        lax.fori_loop(0, GRP // RING, _outer, None)

        if blk == NBLK - 1:
            for k in range(RING):
                pltpu.make_async_copy(dummy16, ring_bufs[k], ring_sems[k]).wait()

    pltpu.sync_copy(den_v, den_sh.at[sid])
    plsc.subcore_barrier()

    sbase = sid * STRIPE
    for t in range(NS):
        pltpu.sync_copy(den_sh.at[t, pl.ds(sbase, STRIPE)], dsum_v.at[t])

    def _red(g, _):
        sl = pl.ds(g * L, L)
        acc = dsum_v[0, sl]
        for t in range(1, NS):
            acc = acc + dsum_v[t, sl]
        dsum_v[0, sl] = acc
        return _
    lax.fori_loop(0, STRIPE // L, _red, None)

    pltpu.sync_copy(dsum_v.at[0], den_out.at[cid, pl.ds(sbase, STRIPE)])
    pltpu.sync_copy(s_sh.at[pl.ds(sbase, STRIPE)],
                    s_out.at[cid, pl.ds(sbase, STRIPE), :])


def _sc_agg(src, dst, a_src, a_dst, beta, e):
    mesh = plsc.VectorSubcoreMesh(core_axis_name="c", subcore_axis_name="s",
                                  num_cores=NC, num_subcores=NS)
    f32 = jnp.float32
    kern = pl.kernel(
        _sc_body,
        out_type=[
            jax.ShapeDtypeStruct((NC, NPAD, ED), f32),
            jax.ShapeDtypeStruct((NC, NPAD), f32),
        ],
        mesh=mesh,
        compiler_params=pltpu.CompilerParams(needs_layout_passes=False,
                                             use_tc_tiling_on_sc=False),
        scratch_types=[
            pltpu.VMEM((N,), f32),
            pltpu.VMEM((N,), f32),
            pltpu.VMEM((NPAD,), f32),
            pltpu.VMEM((BLK,), jnp.int32),
            pltpu.VMEM((BLK,), jnp.int32),
            pltpu.VMEM((BLK,), f32),
            pltpu.VMEM((BLK, ED), f32),
            pltpu.VMEM((L, ED), f32),
            pltpu.VMEM((L, ED), f32),
            pltpu.VMEM((L, ED), f32),
            pltpu.VMEM((L, ED), f32),
            pltpu.VMEM((L, ED), f32),
            pltpu.VMEM((STRIPE, ED), f32),
            pltpu.VMEM((NS, STRIPE), f32),
            pltpu.VMEM_SHARED((NPAD, ED), f32),
            pltpu.VMEM_SHARED((NS, NPAD), f32),
            pltpu.SemaphoreType.DMA,
            pltpu.SemaphoreType.DMA,
            pltpu.SemaphoreType.DMA,
            pltpu.SemaphoreType.DMA,
            pltpu.SemaphoreType.DMA,
            pltpu.SemaphoreType.DMA,
        ],
    )
    return kern(src, dst, a_src, a_dst, beta, e)


def _finish_body(s_ref, d_ref, we_ref, wn_ref, o_ref):
    s = s_ref[0] + s_ref[1]
    d = d_ref[0] + d_ref[1]
    d = jnp.where(d == 0.0, 1.0, d)
    sw = s / d[:, None]
    t = lax.dot_general(sw, we_ref[...], (((1,), (1,)), ((), ())),
                        preferred_element_type=jnp.float32)
    o_ref[...] = lax.dot_general(t, wn_ref[...], (((1,), (1,)), ((), ())),
                                 preferred_element_type=jnp.float32)


def _finish(s_parts, den_parts, W_edge, W_e2n):
    blk = 1024
    grid = (NPAD // blk,)
    return pl.pallas_call(
        _finish_body,
        grid=grid,
        in_specs=[
            pl.BlockSpec((NC, blk, ED), lambda i: (0, i, 0)),
            pl.BlockSpec((NC, blk), lambda i: (0, i)),
            pl.BlockSpec((ED, ED), lambda i: (0, 0)),
            pl.BlockSpec((D, ED), lambda i: (0, 0)),
        ],
        out_specs=pl.BlockSpec((blk, D), lambda i: (i, 0)),
        out_shape=jax.ShapeDtypeStruct((NPAD, D), jnp.float32),
    )(s_parts, den_parts, W_edge, W_e2n)


@jax.jit
def kernel(h, edge_index, e, W_att, W_edge, W_e2n):
    a_src, a_dst = _node_alpha(h, W_att)
    src, dst, beta = _edge_lite(edge_index, e, W_edge, W_att)
    s_parts, den_parts = _sc_agg(src, dst, a_src, a_dst, beta, e)
    return _finish(s_parts, den_parts, W_edge, W_e2n)[:N]


# speedup vs baseline: 1.5609x; 1.5609x over previous
import jax
import jax.numpy as jnp
from jax import lax
from jax.experimental import pallas as pl
from jax.experimental.pallas import tpu as pltpu
from jax.experimental.pallas import tpu_sc as plsc

N = 10000
E = 320000
D = 128
ED = 16

NC = 2
NS = 16
NW = NC * NS
L = 16

EPW = E // NW
BLK = 2000
NBLK = EPW // BLK
GRP = BLK // L
RING = 5

NPAD = 10240
STRIPE = NPAD // NS

EB = 3200


def _node_alpha_body(h_ref, wa_ref, o1_ref, o2_ref):
    hb = h_ref[...]
    w1 = wa_ref[0, :D]
    w2 = wa_ref[0, D:2 * D]
    o1_ref[...] = jnp.dot(hb, w1, preferred_element_type=jnp.float32)
    o2_ref[...] = jnp.dot(hb, w2, preferred_element_type=jnp.float32)


def _node_alpha(h, W_att):
    return pl.pallas_call(
        _node_alpha_body,
        out_shape=[
            jax.ShapeDtypeStruct((N,), jnp.float32),
            jax.ShapeDtypeStruct((N,), jnp.float32),
        ],
    )(h, W_att)


def _edge_lite_body(ei_ref, et_ref, we_ref, wa_ref, src_ref, dst_ref, b_ref):
    i = pl.program_id(0)
    sl = pl.ds(i * EB, EB)
    src_ref[sl] = ei_ref[0, :]
    dst_ref[sl] = ei_ref[1, :]
    w3 = wa_ref[0, 2 * D:]
    v3 = lax.dot_general(w3, we_ref[...], (((0,), (0,)), ((), ())),
                         preferred_element_type=jnp.float32)
    b_ref[sl] = lax.dot_general(v3, et_ref[...], (((0,), (0,)), ((), ())),
                                preferred_element_type=jnp.float32)


def _edge_lite(edge_index, et, W_edge, W_att):
    return pl.pallas_call(
        _edge_lite_body,
        grid=(E // EB,),
        in_specs=[
            pl.BlockSpec((2, EB), lambda i: (0, i)),
            pl.BlockSpec((ED, EB), lambda i: (0, i)),
            pl.BlockSpec((ED, ED), lambda i: (0, 0)),
            pl.BlockSpec((1, 2 * D + ED), lambda i: (0, 0)),
        ],
        out_specs=[
            pl.BlockSpec((E,), lambda i: (0,)),
            pl.BlockSpec((E,), lambda i: (0,)),
            pl.BlockSpec((E,), lambda i: (0,)),
        ],
        out_shape=[
            jax.ShapeDtypeStruct((E,), jnp.int32),
            jax.ShapeDtypeStruct((E,), jnp.int32),
            jax.ShapeDtypeStruct((E,), jnp.float32),
        ],
    )(edge_index, et, W_edge, W_att)


def _sc_body(src_hbm, dst_hbm, asrc_hbm, adst_hbm, beta_hbm, e_hbm,
             s_out, den_out,
             asrc_v, adst_v, den_v, src_v, dstf_v, beta_v, e_v,
             rows0, rows1, rows2, rows3, rows4, zbuf_v, dsum_v,
             s_sh, den_sh, insem, sem0, sem1, sem2, sem3, sem4):
    cid = lax.axis_index("c")
    sid = lax.axis_index("s")
    wid = cid * NS + sid
    ebase = wid * EPW

    ring_bufs = (rows0, rows1, rows2, rows3, rows4)
    ring_sems = (sem0, sem1, sem2, sem3, sem4)
    zero16 = jnp.zeros((L,), jnp.float32)
    dummy16 = e_hbm.at[pl.ds(0, L), pl.ds(0, ED)]

    pltpu.async_copy(asrc_hbm, asrc_v, insem)
    pltpu.async_copy(adst_hbm, adst_v, insem)
    pltpu.make_async_copy(asrc_hbm, asrc_v, insem).wait()
    pltpu.make_async_copy(adst_hbm, adst_v, insem).wait()

    def _z_buf(i, _):
        zbuf_v[i, :] = zero16
        return _
    lax.fori_loop(0, STRIPE, _z_buf, None)

    def _z_den(i, _):
        den_v[pl.ds(i * L, L)] = zero16
        return _
    lax.fori_loop(0, NPAD // L, _z_den, None)

    pltpu.sync_copy(zbuf_v, s_sh.at[pl.ds(sid * STRIPE, STRIPE)])
    plsc.subcore_barrier()

    for blk in range(NBLK):
        base = ebase + blk * BLK
        pltpu.async_copy(src_hbm.at[pl.ds(base, BLK)], src_v, insem)
        pltpu.async_copy(dst_hbm.at[pl.ds(base, BLK)], dstf_v, insem)
        pltpu.async_copy(beta_hbm.at[pl.ds(base, BLK)], beta_v, insem)
        pltpu.async_copy(e_hbm.at[pl.ds(base // 8, BLK // 8), :], e_v, insem)
        pltpu.make_async_copy(src_hbm.at[pl.ds(base, BLK)], src_v, insem).wait()
        pltpu.make_async_copy(dst_hbm.at[pl.ds(base, BLK)], dstf_v, insem).wait()
        pltpu.make_async_copy(beta_hbm.at[pl.ds(base, BLK)], beta_v, insem).wait()
        pltpu.make_async_copy(e_hbm.at[pl.ds(base // 8, BLK // 8), :], e_v,
                              insem).wait()

        def _outer(g2, _):
            for k in range(RING):
                g = g2 * RING + k
                o = g * L
                sv = src_v[pl.ds(o, L)]
                dv = dstf_v[pl.ds(o, L)]
                a1 = plsc.load_gather(asrc_v, [sv])
                a2 = plsc.load_gather(adst_v, [dv])
                att = a1 + a2 + beta_v[pl.ds(o, L)]
                att = jnp.maximum(att, att * 0.01)
                ee = jnp.exp(att)
                plsc.addupdate_scatter(den_v, [dv], ee)
                buf, sem = ring_bufs[k], ring_sems[k]
                if blk == 0:
                    @pl.when(g2 >= 1)
                    def _drain():
                        pltpu.make_async_copy(dummy16, buf, sem).wait()
                else:
                    pltpu.make_async_copy(dummy16, buf, sem).wait()
                for j in range(L):
                    buf[j, :] = e_v[2 * g + j // 8,
                                    pl.ds((j % 8) * L, L)] * ee[j]
                pltpu.async_copy(buf, s_sh.at[dv], sem, add=True)
            return _
        lax.fori_loop(0, GRP // RING, _outer, None)

        if blk == NBLK - 1:
            for k in range(RING):
                pltpu.make_async_copy(dummy16, ring_bufs[k], ring_sems[k]).wait()

    pltpu.sync_copy(den_v, den_sh.at[sid])
    plsc.subcore_barrier()

    sbase = sid * STRIPE
    for t in range(NS):
        pltpu.sync_copy(den_sh.at[t, pl.ds(sbase, STRIPE)], dsum_v.at[t])

    def _red(g, _):
        sl = pl.ds(g * L, L)
        acc = dsum_v[0, sl]
        for t in range(1, NS):
            acc = acc + dsum_v[t, sl]
        dsum_v[0, sl] = acc
        return _
    lax.fori_loop(0, STRIPE // L, _red, None)

    pltpu.sync_copy(dsum_v.at[0], den_out.at[cid, pl.ds(sbase, STRIPE)])
    pltpu.sync_copy(s_sh.at[pl.ds(sbase, STRIPE)],
                    s_out.at[cid, pl.ds(sbase, STRIPE), :])


def _sc_agg(src, dst, a_src, a_dst, beta, e):
    mesh = plsc.VectorSubcoreMesh(core_axis_name="c", subcore_axis_name="s",
                                  num_cores=NC, num_subcores=NS)
    f32 = jnp.float32
    kern = pl.kernel(
        _sc_body,
        out_type=[
            jax.ShapeDtypeStruct((NC, NPAD, ED), f32),
            jax.ShapeDtypeStruct((NC, NPAD), f32),
        ],
        mesh=mesh,
        compiler_params=pltpu.CompilerParams(needs_layout_passes=False,
                                             use_tc_tiling_on_sc=False),
        scratch_types=[
            pltpu.VMEM((N,), f32),
            pltpu.VMEM((N,), f32),
            pltpu.VMEM((NPAD,), f32),
            pltpu.VMEM((BLK,), jnp.int32),
            pltpu.VMEM((BLK,), jnp.int32),
            pltpu.VMEM((BLK,), f32),
            pltpu.VMEM((BLK // 8, 8 * ED), f32),
            pltpu.VMEM((L, ED), f32),
            pltpu.VMEM((L, ED), f32),
            pltpu.VMEM((L, ED), f32),
            pltpu.VMEM((L, ED), f32),
            pltpu.VMEM((L, ED), f32),
            pltpu.VMEM((STRIPE, ED), f32),
            pltpu.VMEM((NS, STRIPE), f32),
            pltpu.VMEM_SHARED((NPAD, ED), f32),
            pltpu.VMEM_SHARED((NS, NPAD), f32),
            pltpu.SemaphoreType.DMA,
            pltpu.SemaphoreType.DMA,
            pltpu.SemaphoreType.DMA,
            pltpu.SemaphoreType.DMA,
            pltpu.SemaphoreType.DMA,
            pltpu.SemaphoreType.DMA,
        ],
    )
    return kern(src, dst, a_src, a_dst, beta, e)


def _finish_body(s_ref, d_ref, we_ref, wn_ref, o_ref):
    s = s_ref[0] + s_ref[1]
    d = d_ref[0] + d_ref[1]
    d = jnp.where(d == 0.0, 1.0, d)
    sw = s / d[:, None]
    t = lax.dot_general(sw, we_ref[...], (((1,), (1,)), ((), ())),
                        preferred_element_type=jnp.float32)
    o_ref[...] = lax.dot_general(t, wn_ref[...], (((1,), (1,)), ((), ())),
                                 preferred_element_type=jnp.float32)


def _finish(s_parts, den_parts, W_edge, W_e2n):
    blk = 1024
    grid = (NPAD // blk,)
    return pl.pallas_call(
        _finish_body,
        grid=grid,
        in_specs=[
            pl.BlockSpec((NC, blk, ED), lambda i: (0, i, 0)),
            pl.BlockSpec((NC, blk), lambda i: (0, i)),
            pl.BlockSpec((ED, ED), lambda i: (0, 0)),
            pl.BlockSpec((D, ED), lambda i: (0, 0)),
        ],
        out_specs=pl.BlockSpec((blk, D), lambda i: (i, 0)),
        out_shape=jax.ShapeDtypeStruct((NPAD, D), jnp.float32),
    )(s_parts, den_parts, W_edge, W_e2n)


@jax.jit
def kernel(h, edge_index, e, W_att, W_edge, W_e2n):
    e8 = e.reshape(E // 8, 8 * ED)
    et = e.T
    a_src, a_dst = _node_alpha(h, W_att)
    src, dst, beta = _edge_lite(edge_index, et, W_edge, W_att)
    s_parts, den_parts = _sc_agg(src, dst, a_src, a_dst, beta, e8)
    return _finish(s_parts, den_parts, W_edge, W_e2n)[:N]


# speedup vs baseline: 2.1508x; 1.3779x over previous
import jax
import jax.numpy as jnp
from jax import lax
from jax.experimental import pallas as pl
from jax.experimental.pallas import tpu as pltpu
from jax.experimental.pallas import tpu_sc as plsc

N = 10000
E = 320000
D = 128
ED = 16

NC = 2
NS = 16
NW = NC * NS
L = 16

EPW = E // NW
BLK = 2000
NBLK = EPW // BLK
GRP = BLK // L

EPC = E // NC
BLK2 = 8000
NBLK2 = EPC // BLK2
GRP2 = BLK2 // L

NPAD = 10240
STRIPE = NPAD // NS

EB = 3200


def _node_alpha_body(h_ref, wa_ref, o1_ref, o2_ref):
    hb = h_ref[...]
    w1 = wa_ref[0, :D]
    w2 = wa_ref[0, D:2 * D]
    o1_ref[...] = jnp.dot(hb, w1, preferred_element_type=jnp.float32)
    o2_ref[...] = jnp.dot(hb, w2, preferred_element_type=jnp.float32)


def _node_alpha(h, W_att):
    return pl.pallas_call(
        _node_alpha_body,
        out_shape=[
            jax.ShapeDtypeStruct((N,), jnp.float32),
            jax.ShapeDtypeStruct((N,), jnp.float32),
        ],
    )(h, W_att)


def _edge_lite_body(ei_ref, et_ref, we_ref, wa_ref, src_ref, dst_ref, b_ref):
    i = pl.program_id(0)
    sl = pl.ds(i * EB, EB)
    src_ref[sl] = ei_ref[0, :]
    dst_ref[sl] = ei_ref[1, :]
    w3 = wa_ref[0, 2 * D:]
    v3 = lax.dot_general(w3, we_ref[...], (((0,), (0,)), ((), ())),
                         preferred_element_type=jnp.float32)
    b_ref[sl] = lax.dot_general(v3, et_ref[...], (((0,), (0,)), ((), ())),
                                preferred_element_type=jnp.float32)


def _edge_lite(edge_index, et, W_edge, W_att):
    return pl.pallas_call(
        _edge_lite_body,
        grid=(E // EB,),
        in_specs=[
            pl.BlockSpec((2, EB), lambda i: (0, i)),
            pl.BlockSpec((ED, EB), lambda i: (0, i)),
            pl.BlockSpec((ED, ED), lambda i: (0, 0)),
            pl.BlockSpec((1, 2 * D + ED), lambda i: (0, 0)),
        ],
        out_specs=[
            pl.BlockSpec((E,), lambda i: (0,)),
            pl.BlockSpec((E,), lambda i: (0,)),
            pl.BlockSpec((E,), lambda i: (0,)),
        ],
        out_shape=[
            jax.ShapeDtypeStruct((E,), jnp.int32),
            jax.ShapeDtypeStruct((E,), jnp.int32),
            jax.ShapeDtypeStruct((E,), jnp.float32),
        ],
    )(edge_index, et, W_edge, W_att)


def _sc_body(src_hbm, dst_hbm, asrc_hbm, adst_hbm, beta_hbm, et_hbm,
             s_out, den_out, ee_out,
             asrc_v, adst_v, den_v, sf_v, src_v, dstf_v, beta_v, ee_v,
             dst2a, dst2b, ee2a, ee2b, efa, efb, dsum_v,
             den_sh, insem, psa, psb):
    cid = lax.axis_index("c")
    sid = lax.axis_index("s")
    wid = cid * NS + sid
    ebase = wid * EPW

    zero16 = jnp.zeros((L,), jnp.float32)

    pltpu.async_copy(asrc_hbm, asrc_v, insem)
    pltpu.async_copy(adst_hbm, adst_v, insem)
    pltpu.make_async_copy(asrc_hbm, asrc_v, insem).wait()
    pltpu.make_async_copy(adst_hbm, adst_v, insem).wait()

    def _z_den(i, _):
        den_v[pl.ds(i * L, L)] = zero16
        sf_v[pl.ds(i * L, L)] = zero16
        return _
    lax.fori_loop(0, NPAD // L, _z_den, None)

    for blk in range(NBLK):
        base = ebase + blk * BLK
        pltpu.async_copy(src_hbm.at[pl.ds(base, BLK)], src_v, insem)
        pltpu.async_copy(dst_hbm.at[pl.ds(base, BLK)], dstf_v, insem)
        pltpu.async_copy(beta_hbm.at[pl.ds(base, BLK)], beta_v, insem)
        pltpu.make_async_copy(src_hbm.at[pl.ds(base, BLK)], src_v, insem).wait()
        pltpu.make_async_copy(dst_hbm.at[pl.ds(base, BLK)], dstf_v, insem).wait()
        pltpu.make_async_copy(beta_hbm.at[pl.ds(base, BLK)], beta_v, insem).wait()

        def _grp(g, _):
            o = g * L
            sv = src_v[pl.ds(o, L)]
            dv = dstf_v[pl.ds(o, L)]
            a1 = plsc.load_gather(asrc_v, [sv])
            a2 = plsc.load_gather(adst_v, [dv])
            att = a1 + a2 + beta_v[pl.ds(o, L)]
            att = jnp.maximum(att, att * 0.01)
            ee = jnp.exp(att)
            plsc.addupdate_scatter(den_v, [dv], ee)
            ee_v[pl.ds(o, L)] = ee
            return _
        lax.fori_loop(0, GRP, _grp, None)
        pltpu.sync_copy(ee_v, ee_out.at[pl.ds(base, BLK)])

    plsc.subcore_barrier()

    dst_bufs = (dst2a, dst2b)
    ee_bufs = (ee2a, ee2b)
    ef_bufs = (efa, efb)
    sems = (psa, psb)
    cbase = cid * EPC

    def _issue(b2, which):
        b = cbase + b2 * BLK2
        pltpu.async_copy(dst_hbm.at[pl.ds(b, BLK2)], dst_bufs[which], sems[which])
        pltpu.async_copy(ee_out.at[pl.ds(b, BLK2)], ee_bufs[which], sems[which])
        pltpu.async_copy(et_hbm.at[sid, pl.ds(b, BLK2)], ef_bufs[which],
                         sems[which])

    def _wait(b2, which):
        b = cbase + b2 * BLK2
        pltpu.make_async_copy(dst_hbm.at[pl.ds(b, BLK2)], dst_bufs[which],
                              sems[which]).wait()
        pltpu.make_async_copy(ee_out.at[pl.ds(b, BLK2)], ee_bufs[which],
                              sems[which]).wait()
        pltpu.make_async_copy(et_hbm.at[sid, pl.ds(b, BLK2)], ef_bufs[which],
                              sems[which]).wait()

    _issue(0, 0)
    for b2 in range(NBLK2):
        which = b2 % 2
        _wait(b2, which)
        if b2 + 1 < NBLK2:
            _issue(b2 + 1, 1 - which)
        dbuf, ebuf, fbuf = dst_bufs[which], ee_bufs[which], ef_bufs[which]

        def _grp2(g, _):
            o = g * L
            dv = dbuf[pl.ds(o, L)]
            val = ebuf[pl.ds(o, L)] * fbuf[pl.ds(o, L)]
            plsc.addupdate_scatter(sf_v, [dv], val)
            return _
        lax.fori_loop(0, GRP2, _grp2, None)

    pltpu.sync_copy(sf_v, s_out.at[cid, sid, :])

    pltpu.sync_copy(den_v, den_sh.at[sid])
    plsc.subcore_barrier()

    sbase = sid * STRIPE
    for t in range(NS):
        pltpu.sync_copy(den_sh.at[t, pl.ds(sbase, STRIPE)], dsum_v.at[t])

    def _red(g, _):
        sl = pl.ds(g * L, L)
        acc = dsum_v[0, sl]
        for t in range(1, NS):
            acc = acc + dsum_v[t, sl]
        dsum_v[0, sl] = acc
        return _
    lax.fori_loop(0, STRIPE // L, _red, None)

    pltpu.sync_copy(dsum_v.at[0], den_out.at[cid, pl.ds(sbase, STRIPE)])


def _sc_agg(src, dst, a_src, a_dst, beta, et):
    mesh = plsc.VectorSubcoreMesh(core_axis_name="c", subcore_axis_name="s",
                                  num_cores=NC, num_subcores=NS)
    f32 = jnp.float32
    kern = pl.kernel(
        _sc_body,
        out_type=[
            jax.ShapeDtypeStruct((NC, ED, NPAD), f32),
            jax.ShapeDtypeStruct((NC, NPAD), f32),
            jax.ShapeDtypeStruct((E,), f32),
        ],
        mesh=mesh,
        compiler_params=pltpu.CompilerParams(needs_layout_passes=False,
                                             use_tc_tiling_on_sc=False),
        scratch_types=[
            pltpu.VMEM((N,), f32),
            pltpu.VMEM((N,), f32),
            pltpu.VMEM((NPAD,), f32),
            pltpu.VMEM((NPAD,), f32),
            pltpu.VMEM((BLK,), jnp.int32),
            pltpu.VMEM((BLK,), jnp.int32),
            pltpu.VMEM((BLK,), f32),
            pltpu.VMEM((BLK,), f32),
            pltpu.VMEM((BLK2,), jnp.int32),
            pltpu.VMEM((BLK2,), jnp.int32),
            pltpu.VMEM((BLK2,), f32),
            pltpu.VMEM((BLK2,), f32),
            pltpu.VMEM((BLK2,), f32),
            pltpu.VMEM((BLK2,), f32),
            pltpu.VMEM((NS, STRIPE), f32),
            pltpu.VMEM_SHARED((NS, NPAD), f32),
            pltpu.SemaphoreType.DMA,
            pltpu.SemaphoreType.DMA,
            pltpu.SemaphoreType.DMA,
        ],
    )
    return kern(src, dst, a_src, a_dst, beta, et)


def _finish_body(s_ref, d_ref, we_ref, wn_ref, o_ref):
    s = s_ref[0] + s_ref[1]
    d = d_ref[0] + d_ref[1]
    d = jnp.where(d == 0.0, 1.0, d)
    sw = s / d[None, :]
    t = lax.dot_general(we_ref[...], sw, (((1,), (0,)), ((), ())),
                        preferred_element_type=jnp.float32)
    o_ref[...] = lax.dot_general(t, wn_ref[...], (((0,), (1,)), ((), ())),
                                 preferred_element_type=jnp.float32)


def _finish(s_parts, den_parts, W_edge, W_e2n):
    blk = 1024
    grid = (NPAD // blk,)
    return pl.pallas_call(
        _finish_body,
        grid=grid,
        in_specs=[
            pl.BlockSpec((NC, ED, blk), lambda i: (0, 0, i)),
            pl.BlockSpec((NC, blk), lambda i: (0, i)),
            pl.BlockSpec((ED, ED), lambda i: (0, 0)),
            pl.BlockSpec((D, ED), lambda i: (0, 0)),
        ],
        out_specs=pl.BlockSpec((blk, D), lambda i: (i, 0)),
        out_shape=jax.ShapeDtypeStruct((NPAD, D), jnp.float32),
    )(s_parts, den_parts, W_edge, W_e2n)


@jax.jit
def kernel(h, edge_index, e, W_att, W_edge, W_e2n):
    et = e.T
    a_src, a_dst = _node_alpha(h, W_att)
    src, dst, beta = _edge_lite(edge_index, et, W_edge, W_att)
    s_parts, den_parts, _ = _sc_agg(src, dst, a_src, a_dst, beta, et)
    return _finish(s_parts, den_parts, W_edge, W_e2n)[:N]


# speedup vs baseline: 2.5427x; 1.1822x over previous
import jax
import jax.numpy as jnp
from jax import lax
from jax.experimental import pallas as pl
from jax.experimental.pallas import tpu as pltpu
from jax.experimental.pallas import tpu_sc as plsc

N = 10000
E = 320000
D = 128
ED = 16

NC = 2
NS = 16
NW = NC * NS
L = 16

EPW = E // NW
BLK = 2000
NBLK = EPW // BLK
GRP = BLK // L

EPC = E // NC
BLK2 = 8000
NBLK2 = EPC // BLK2
GRP2 = BLK2 // L

NPAD = 10240
STRIPE = NPAD // NS

EB = 3200


def _node_alpha_body(h_ref, wa_ref, o1_ref, o2_ref):
    hb = h_ref[...]
    w1 = wa_ref[0, :D]
    w2 = wa_ref[0, D:2 * D]
    o1_ref[...] = jnp.dot(hb, w1, preferred_element_type=jnp.float32)
    o2_ref[...] = jnp.dot(hb, w2, preferred_element_type=jnp.float32)


def _node_alpha(h, W_att):
    return pl.pallas_call(
        _node_alpha_body,
        out_shape=[
            jax.ShapeDtypeStruct((N,), jnp.float32),
            jax.ShapeDtypeStruct((N,), jnp.float32),
        ],
    )(h, W_att)


EB = 16000


def _split_body(ei_ref, src_ref, dst_ref):
    i = pl.program_id(0)
    sl = pl.ds(i * EB, EB)
    src_ref[sl] = ei_ref[0, :]
    dst_ref[sl] = ei_ref[1, :]


def _split_ei(edge_index):
    return pl.pallas_call(
        _split_body,
        grid=(E // EB,),
        in_specs=[pl.BlockSpec((2, EB), lambda i: (0, i))],
        out_specs=[
            pl.BlockSpec((E,), lambda i: (0,)),
            pl.BlockSpec((E,), lambda i: (0,)),
        ],
        out_shape=[
            jax.ShapeDtypeStruct((E,), jnp.int32),
            jax.ShapeDtypeStruct((E,), jnp.int32),
        ],
    )(edge_index)


def _beta_body(et_ref, we_ref, wa_ref, b_ref):
    i = pl.program_id(0)
    w3 = wa_ref[0, 2 * D:]
    v3 = lax.dot_general(w3, we_ref[...], (((0,), (0,)), ((), ())),
                         preferred_element_type=jnp.float32)
    b_ref[pl.ds(i * EB, EB)] = lax.dot_general(
        v3, et_ref[...], (((0,), (0,)), ((), ())),
        preferred_element_type=jnp.float32)


def _beta_k(et, W_edge, W_att):
    return pl.pallas_call(
        _beta_body,
        grid=(E // EB,),
        in_specs=[
            pl.BlockSpec((ED, EB), lambda i: (0, i)),
            pl.BlockSpec((ED, ED), lambda i: (0, 0)),
            pl.BlockSpec((1, 2 * D + ED), lambda i: (0, 0)),
        ],
        out_specs=pl.BlockSpec((E,), lambda i: (0,)),
        out_shape=jax.ShapeDtypeStruct((E,), jnp.float32),
    )(et, W_edge, W_att)


def _sc_body(src_hbm, dst_hbm, asrc_hbm, adst_hbm, beta_hbm, et_hbm,
             s_out, den_out, ee_out,
             asrc_v, adst_v, den_v, sf_v, src_v, dstf_v, beta_v, ee_v,
             dst2a, dst2b, ee2a, ee2b, efa, efb, dsum_v,
             den_sh, insem, psa, psb):
    cid = lax.axis_index("c")
    sid = lax.axis_index("s")
    wid = cid * NS + sid
    ebase = wid * EPW

    zero16 = jnp.zeros((L,), jnp.float32)

    pltpu.async_copy(asrc_hbm, asrc_v, insem)
    pltpu.async_copy(adst_hbm, adst_v, insem)
    pltpu.make_async_copy(asrc_hbm, asrc_v, insem).wait()
    pltpu.make_async_copy(adst_hbm, adst_v, insem).wait()

    def _z_den(i, _):
        den_v[pl.ds(i * L, L)] = zero16
        sf_v[pl.ds(i * L, L)] = zero16
        return _
    lax.fori_loop(0, NPAD // L, _z_den, None)

    for blk in range(NBLK):
        base = ebase + blk * BLK
        pltpu.async_copy(src_hbm.at[pl.ds(base, BLK)], src_v, insem)
        pltpu.async_copy(dst_hbm.at[pl.ds(base, BLK)], dstf_v, insem)
        pltpu.async_copy(beta_hbm.at[pl.ds(base, BLK)], beta_v, insem)
        pltpu.make_async_copy(src_hbm.at[pl.ds(base, BLK)], src_v, insem).wait()
        pltpu.make_async_copy(dst_hbm.at[pl.ds(base, BLK)], dstf_v, insem).wait()
        pltpu.make_async_copy(beta_hbm.at[pl.ds(base, BLK)], beta_v, insem).wait()

        def _grp(g, _):
            o = g * L
            sv = src_v[pl.ds(o, L)]
            dv = dstf_v[pl.ds(o, L)]
            a1 = plsc.load_gather(asrc_v, [sv])
            a2 = plsc.load_gather(adst_v, [dv])
            att = a1 + a2 + beta_v[pl.ds(o, L)]
            att = jnp.maximum(att, att * 0.01)
            ee = jnp.exp(att)
            plsc.addupdate_scatter(den_v, [dv], ee)
            ee_v[pl.ds(o, L)] = ee
            return _
        lax.fori_loop(0, GRP, _grp, None)
        pltpu.sync_copy(ee_v, ee_out.at[pl.ds(base, BLK)])

    plsc.subcore_barrier()

    dst_bufs = (dst2a, dst2b)
    ee_bufs = (ee2a, ee2b)
    ef_bufs = (efa, efb)
    sems = (psa, psb)
    cbase = cid * EPC

    def _issue(b2, which):
        b = cbase + b2 * BLK2
        pltpu.async_copy(dst_hbm.at[pl.ds(b, BLK2)], dst_bufs[which], sems[which])
        pltpu.async_copy(ee_out.at[pl.ds(b, BLK2)], ee_bufs[which], sems[which])
        pltpu.async_copy(et_hbm.at[sid, pl.ds(b, BLK2)], ef_bufs[which],
                         sems[which])

    def _wait(b2, which):
        b = cbase + b2 * BLK2
        pltpu.make_async_copy(dst_hbm.at[pl.ds(b, BLK2)], dst_bufs[which],
                              sems[which]).wait()
        pltpu.make_async_copy(ee_out.at[pl.ds(b, BLK2)], ee_bufs[which],
                              sems[which]).wait()
        pltpu.make_async_copy(et_hbm.at[sid, pl.ds(b, BLK2)], ef_bufs[which],
                              sems[which]).wait()

    _issue(0, 0)
    for b2 in range(NBLK2):
        which = b2 % 2
        _wait(b2, which)
        if b2 + 1 < NBLK2:
            _issue(b2 + 1, 1 - which)
        dbuf, ebuf, fbuf = dst_bufs[which], ee_bufs[which], ef_bufs[which]

        def _grp2(g, _):
            o = g * L
            dv = dbuf[pl.ds(o, L)]
            val = ebuf[pl.ds(o, L)] * fbuf[pl.ds(o, L)]
            plsc.addupdate_scatter(sf_v, [dv], val)
            return _
        lax.fori_loop(0, GRP2, _grp2, None)

    pltpu.sync_copy(sf_v, s_out.at[cid, sid, :])

    pltpu.sync_copy(den_v, den_sh.at[sid])
    plsc.subcore_barrier()

    sbase = sid * STRIPE
    for t in range(NS):
        pltpu.sync_copy(den_sh.at[t, pl.ds(sbase, STRIPE)], dsum_v.at[t])

    def _red(g, _):
        sl = pl.ds(g * L, L)
        acc = dsum_v[0, sl]
        for t in range(1, NS):
            acc = acc + dsum_v[t, sl]
        dsum_v[0, sl] = acc
        return _
    lax.fori_loop(0, STRIPE // L, _red, None)

    pltpu.sync_copy(dsum_v.at[0], den_out.at[cid, pl.ds(sbase, STRIPE)])


def _sc_agg(src, dst, a_src, a_dst, beta, et):
    mesh = plsc.VectorSubcoreMesh(core_axis_name="c", subcore_axis_name="s",
                                  num_cores=NC, num_subcores=NS)
    f32 = jnp.float32
    kern = pl.kernel(
        _sc_body,
        out_type=[
            jax.ShapeDtypeStruct((NC, ED, NPAD), f32),
            jax.ShapeDtypeStruct((NC, NPAD), f32),
            jax.ShapeDtypeStruct((E,), f32),
        ],
        mesh=mesh,
        compiler_params=pltpu.CompilerParams(needs_layout_passes=False,
                                             use_tc_tiling_on_sc=False),
        scratch_types=[
            pltpu.VMEM((N,), f32),
            pltpu.VMEM((N,), f32),
            pltpu.VMEM((NPAD,), f32),
            pltpu.VMEM((NPAD,), f32),
            pltpu.VMEM((BLK,), jnp.int32),
            pltpu.VMEM((BLK,), jnp.int32),
            pltpu.VMEM((BLK,), f32),
            pltpu.VMEM((BLK,), f32),
            pltpu.VMEM((BLK2,), jnp.int32),
            pltpu.VMEM((BLK2,), jnp.int32),
            pltpu.VMEM((BLK2,), f32),
            pltpu.VMEM((BLK2,), f32),
            pltpu.VMEM((BLK2,), f32),
            pltpu.VMEM((BLK2,), f32),
            pltpu.VMEM((NS, STRIPE), f32),
            pltpu.VMEM_SHARED((NS, NPAD), f32),
            pltpu.SemaphoreType.DMA,
            pltpu.SemaphoreType.DMA,
            pltpu.SemaphoreType.DMA,
        ],
    )
    return kern(src, dst, a_src, a_dst, beta, et)


def _finish_body(s_ref, d_ref, we_ref, wn_ref, o_ref):
    s = s_ref[0] + s_ref[1]
    d = d_ref[0] + d_ref[1]
    d = jnp.where(d == 0.0, 1.0, d)
    sw = s / d[None, :]
    t = lax.dot_general(we_ref[...], sw, (((1,), (0,)), ((), ())),
                        preferred_element_type=jnp.float32)
    o_ref[...] = lax.dot_general(t, wn_ref[...], (((0,), (1,)), ((), ())),
                                 preferred_element_type=jnp.float32)


def _finish(s_parts, den_parts, W_edge, W_e2n):
    blk = 1024
    grid = (NPAD // blk,)
    return pl.pallas_call(
        _finish_body,
        grid=grid,
        in_specs=[
            pl.BlockSpec((NC, ED, blk), lambda i: (0, 0, i)),
            pl.BlockSpec((NC, blk), lambda i: (0, i)),
            pl.BlockSpec((ED, ED), lambda i: (0, 0)),
            pl.BlockSpec((D, ED), lambda i: (0, 0)),
        ],
        out_specs=pl.BlockSpec((blk, D), lambda i: (i, 0)),
        out_shape=jax.ShapeDtypeStruct((NPAD, D), jnp.float32),
    )(s_parts, den_parts, W_edge, W_e2n)


@jax.jit
def kernel(h, edge_index, e, W_att, W_edge, W_e2n):
    et = e.T
    a_src, a_dst = _node_alpha(h, W_att)
    src, dst = _split_ei(edge_index)
    beta = _beta_k(et, W_edge, W_att)
    s_parts, den_parts, _ = _sc_agg(src, dst, a_src, a_dst, beta, et)
    return _finish(s_parts, den_parts, W_edge, W_e2n)[:N]


# speedup vs baseline: 2.5445x; 1.0007x over previous
import jax
import jax.numpy as jnp
from jax import lax
from jax.experimental import pallas as pl
from jax.experimental.pallas import tpu as pltpu
from jax.experimental.pallas import tpu_sc as plsc

N = 10000
E = 320000
D = 128
ED = 16

NC = 2
NS = 16
NW = NC * NS
L = 16

EPW = E // NW
BLK = 2000
NBLK = EPW // BLK
GRP = BLK // L

EPC = E // NC
BLK2 = 8000
NBLK2 = EPC // BLK2
GRP2 = BLK2 // L

NPAD = 10240
STRIPE = NPAD // NS


def _node_alpha_body(h_ref, wa_ref, o1_ref, o2_ref):
    hb = h_ref[...]
    w1 = wa_ref[0, :D]
    w2 = wa_ref[0, D:2 * D]
    o1_ref[...] = jnp.dot(hb, w1, preferred_element_type=jnp.float32)
    o2_ref[...] = jnp.dot(hb, w2, preferred_element_type=jnp.float32)


def _node_alpha(h, W_att):
    return pl.pallas_call(
        _node_alpha_body,
        out_shape=[
            jax.ShapeDtypeStruct((N,), jnp.float32),
            jax.ShapeDtypeStruct((N,), jnp.float32),
        ],
    )(h, W_att)


EB = 16000


def _split_body(ei_ref, src_ref, dst_ref):
    i = pl.program_id(0)
    sl = pl.ds(i * EB, EB)
    src_ref[sl] = ei_ref[0, :]
    dst_ref[sl] = ei_ref[1, :]


def _split_ei(edge_index):
    return pl.pallas_call(
        _split_body,
        grid=(E // EB,),
        in_specs=[pl.BlockSpec((2, EB), lambda i: (0, i))],
        out_specs=[
            pl.BlockSpec((E,), lambda i: (0,)),
            pl.BlockSpec((E,), lambda i: (0,)),
        ],
        out_shape=[
            jax.ShapeDtypeStruct((E,), jnp.int32),
            jax.ShapeDtypeStruct((E,), jnp.int32),
        ],
    )(edge_index)


def _beta_body(et_ref, we_ref, wa_ref, b_ref):
    i = pl.program_id(0)
    w3 = wa_ref[0, 2 * D:]
    v3 = lax.dot_general(w3, we_ref[...], (((0,), (0,)), ((), ())),
                         preferred_element_type=jnp.float32)
    b_ref[pl.ds(i * EB, EB)] = lax.dot_general(
        v3, et_ref[...], (((0,), (0,)), ((), ())),
        preferred_element_type=jnp.float32)


def _beta_k(et, W_edge, W_att):
    return pl.pallas_call(
        _beta_body,
        grid=(E // EB,),
        in_specs=[
            pl.BlockSpec((ED, EB), lambda i: (0, i)),
            pl.BlockSpec((ED, ED), lambda i: (0, 0)),
            pl.BlockSpec((1, 2 * D + ED), lambda i: (0, 0)),
        ],
        out_specs=pl.BlockSpec((E,), lambda i: (0,)),
        out_shape=jax.ShapeDtypeStruct((E,), jnp.float32),
    )(et, W_edge, W_att)


def _sc_body(src_hbm, dst_hbm, asrc_hbm, adst_hbm, beta_hbm, et_hbm,
             s_out, den_out, ee_out,
             asrc_v, adst_v, den_v, sf_v, src_v, dstf_v, beta_v, ee_v,
             dst2a, dst2b, ee2a, ee2b, efa, efb, dsum_v,
             den_sh, insem, psa, psb):
    cid = lax.axis_index("c")
    sid = lax.axis_index("s")
    wid = cid * NS + sid
    ebase = wid * EPW

    zero16 = jnp.zeros((L,), jnp.float32)

    pltpu.async_copy(asrc_hbm, asrc_v, insem)
    pltpu.async_copy(adst_hbm, adst_v, insem)
    pltpu.make_async_copy(asrc_hbm, asrc_v, insem).wait()
    pltpu.make_async_copy(adst_hbm, adst_v, insem).wait()

    def _z_den(i, _):
        den_v[pl.ds(i * L, L)] = zero16
        sf_v[pl.ds(i * L, L)] = zero16
        return _
    lax.fori_loop(0, NPAD // L, _z_den, None)

    for blk in range(NBLK):
        base = ebase + blk * BLK
        pltpu.async_copy(src_hbm.at[pl.ds(base, BLK)], src_v, insem)
        pltpu.async_copy(dst_hbm.at[pl.ds(base, BLK)], dstf_v, insem)
        pltpu.async_copy(beta_hbm.at[pl.ds(base, BLK)], beta_v, insem)
        pltpu.make_async_copy(src_hbm.at[pl.ds(base, BLK)], src_v, insem).wait()
        pltpu.make_async_copy(dst_hbm.at[pl.ds(base, BLK)], dstf_v, insem).wait()
        pltpu.make_async_copy(beta_hbm.at[pl.ds(base, BLK)], beta_v, insem).wait()

        def _grp(g, _):
            o = g * L
            sv = src_v[pl.ds(o, L)]
            dv = dstf_v[pl.ds(o, L)]
            a1 = plsc.load_gather(asrc_v, [sv])
            a2 = plsc.load_gather(adst_v, [dv])
            att = a1 + a2 + beta_v[pl.ds(o, L)]
            att = jnp.maximum(att, att * 0.01)
            ee = jnp.exp(att)
            plsc.addupdate_scatter(den_v, [dv], ee)
            ee_v[pl.ds(o, L)] = ee
            return _
        lax.fori_loop(0, GRP, _grp, None)
        pltpu.sync_copy(ee_v, ee_out.at[pl.ds(base, BLK)])

    plsc.subcore_barrier()

    dst_bufs = (dst2a, dst2b)
    ee_bufs = (ee2a, ee2b)
    ef_bufs = (efa, efb)
    sems = (psa, psb)
    cbase = cid * EPC

    def _issue(b2, which):
        b = cbase + b2 * BLK2
        pltpu.async_copy(dst_hbm.at[pl.ds(b, BLK2)], dst_bufs[which], sems[which])
        pltpu.async_copy(ee_out.at[pl.ds(b, BLK2)], ee_bufs[which], sems[which])
        pltpu.async_copy(et_hbm.at[sid, pl.ds(b, BLK2)], ef_bufs[which],
                         sems[which])

    def _wait(b2, which):
        b = cbase + b2 * BLK2
        pltpu.make_async_copy(dst_hbm.at[pl.ds(b, BLK2)], dst_bufs[which],
                              sems[which]).wait()
        pltpu.make_async_copy(ee_out.at[pl.ds(b, BLK2)], ee_bufs[which],
                              sems[which]).wait()
        pltpu.make_async_copy(et_hbm.at[sid, pl.ds(b, BLK2)], ef_bufs[which],
                              sems[which]).wait()

    _issue(0, 0)
    for b2 in range(NBLK2):
        which = b2 % 2
        _wait(b2, which)
        if b2 + 1 < NBLK2:
            _issue(b2 + 1, 1 - which)
        dbuf, ebuf, fbuf = dst_bufs[which], ee_bufs[which], ef_bufs[which]

        def _grp2(g, _):
            o = g * L
            dv = dbuf[pl.ds(o, L)]
            val = ebuf[pl.ds(o, L)] * fbuf[pl.ds(o, L)]
            plsc.addupdate_scatter(sf_v, [dv], val)
            return _
        lax.fori_loop(0, GRP2, _grp2, None)

    pltpu.sync_copy(sf_v, s_out.at[cid, sid, :])

    pltpu.sync_copy(den_v, den_sh.at[sid])
    plsc.subcore_barrier()

    sbase = sid * STRIPE
    for t in range(NS):
        pltpu.sync_copy(den_sh.at[t, pl.ds(sbase, STRIPE)], dsum_v.at[t])

    def _red(g, _):
        sl = pl.ds(g * L, L)
        acc = dsum_v[0, sl]
        for t in range(1, NS):
            acc = acc + dsum_v[t, sl]
        dsum_v[0, sl] = acc
        return _
    lax.fori_loop(0, STRIPE // L, _red, None)

    pltpu.sync_copy(dsum_v.at[0], den_out.at[cid, pl.ds(sbase, STRIPE)])


def _sc_agg(src, dst, a_src, a_dst, beta, et):
    mesh = plsc.VectorSubcoreMesh(core_axis_name="c", subcore_axis_name="s",
                                  num_cores=NC, num_subcores=NS)
    f32 = jnp.float32
    kern = pl.kernel(
        _sc_body,
        out_type=[
            jax.ShapeDtypeStruct((NC, ED, NPAD), f32),
            jax.ShapeDtypeStruct((NC, NPAD), f32),
            jax.ShapeDtypeStruct((E,), f32),
        ],
        mesh=mesh,
        compiler_params=pltpu.CompilerParams(needs_layout_passes=False,
                                             use_tc_tiling_on_sc=False),
        scratch_types=[
            pltpu.VMEM((N,), f32),
            pltpu.VMEM((N,), f32),
            pltpu.VMEM((NPAD,), f32),
            pltpu.VMEM((NPAD,), f32),
            pltpu.VMEM((BLK,), jnp.int32),
            pltpu.VMEM((BLK,), jnp.int32),
            pltpu.VMEM((BLK,), f32),
            pltpu.VMEM((BLK,), f32),
            pltpu.VMEM((BLK2,), jnp.int32),
            pltpu.VMEM((BLK2,), jnp.int32),
            pltpu.VMEM((BLK2,), f32),
            pltpu.VMEM((BLK2,), f32),
            pltpu.VMEM((BLK2,), f32),
            pltpu.VMEM((BLK2,), f32),
            pltpu.VMEM((NS, STRIPE), f32),
            pltpu.VMEM_SHARED((NS, NPAD), f32),
            pltpu.SemaphoreType.DMA,
            pltpu.SemaphoreType.DMA,
            pltpu.SemaphoreType.DMA,
        ],
    )
    return kern(src, dst, a_src, a_dst, beta, et)


def _finish_body(s_ref, d_ref, we_ref, wn_ref, o_ref):
    s = s_ref[0] + s_ref[1]
    d = d_ref[0] + d_ref[1]
    d = jnp.where(d == 0.0, 1.0, d)
    sw = s / d[None, :]
    t = lax.dot_general(we_ref[...], sw, (((1,), (0,)), ((), ())),
                        preferred_element_type=jnp.float32)
    o_ref[...] = lax.dot_general(t, wn_ref[...], (((0,), (1,)), ((), ())),
                                 preferred_element_type=jnp.float32)


def _finish(s_parts, den_parts, W_edge, W_e2n):
    blk = 1024
    grid = (NPAD // blk,)
    return pl.pallas_call(
        _finish_body,
        grid=grid,
        in_specs=[
            pl.BlockSpec((NC, ED, blk), lambda i: (0, 0, i)),
            pl.BlockSpec((NC, blk), lambda i: (0, i)),
            pl.BlockSpec((ED, ED), lambda i: (0, 0)),
            pl.BlockSpec((D, ED), lambda i: (0, 0)),
        ],
        out_specs=pl.BlockSpec((blk, D), lambda i: (i, 0)),
        out_shape=jax.ShapeDtypeStruct((NPAD, D), jnp.float32),
    )(s_parts, den_parts, W_edge, W_e2n)


@jax.jit
def kernel(h, edge_index, e, W_att, W_edge, W_e2n):
    et = e.T
    a_src, a_dst = _node_alpha(h, W_att)
    src, dst = _split_ei(edge_index)
    beta = _beta_k(et, W_edge, W_att)
    s_parts, den_parts, _ = _sc_agg(src, dst, a_src, a_dst, beta, et)
    return _finish(s_parts, den_parts, W_edge, W_e2n)[:N]
